# baseline (speedup 1.0000x reference)
import functools

import jax
import jax.numpy as jnp
from jax import lax
from jax.experimental import pallas as pl
from jax.experimental.pallas import tpu as pltpu
from jax.experimental.pallas import tpu_sc as plsc

N = 10000
E = 320000
D = 128
ALPHA = 0.2

NP = 10112
NC, NS = 2, 16
NW = NC * NS
CHUNK = 96
EPT = 10080
NCHUNK = EPT // CHUNK
RPT = NP // NS
DROW = 10100
DNP = 10240

_f32 = jnp.float32


def _mm_body(x_ref, w_ref, arow_ref, acol_ref, h_ref, ar_ref, ac_ref):
    h = jnp.dot(x_ref[...], w_ref[...], preferred_element_type=_f32)
    h_ref[...] = h
    ar_ref[...] = jnp.dot(h, arow_ref[...], preferred_element_type=_f32)[:, 0]
    ac_ref[...] = jnp.dot(h, acol_ref[...], preferred_element_type=_f32)[:, 0]


_MMBLK = 1024
_CBLK = 128


_XBLK = 1000


def _mm_call(x, W, a_row, a_col):
    return pl.pallas_call(
        _mm_body,
        grid=(10,),
        in_specs=[
            pl.BlockSpec((_MMBLK, D), lambda i: (i, 0)),
            pl.BlockSpec((D, D), lambda i: (0, 0)),
            pl.BlockSpec((D, 1), lambda i: (0, 0)),
            pl.BlockSpec((D, 1), lambda i: (0, 0)),
        ],
        out_specs=[
            pl.BlockSpec((_MMBLK, D), lambda i: (i, 0)),
            pl.BlockSpec((_MMBLK,), lambda i: (i,)),
            pl.BlockSpec((_MMBLK,), lambda i: (i,)),
        ],
        out_shape=[
            jax.ShapeDtypeStruct((10240, D), _f32),
            jax.ShapeDtypeStruct((10240,), _f32),
            jax.ShapeDtypeStruct((10240,), _f32),
        ],
    )(x, W, a_row, a_col)


def _sc_body(hx_hbm, ar_hbm, ac_hbm, pk_hbm,
             num_hbm, den_hbm,
             num_sh, den_sh,
             pk_all,
             rv0, rv1, rv2, cv0, cv1, cv2,
             av0, av1, av2, bv0, bv1, bv2,
             h0, h1, h2, zbuf,
             sg0, sg1, sg2, ss0, ss1, ss2):
    rv = (rv0, rv1, rv2)
    cv = (cv0, cv1, cv2)
    av = (av0, av1, av2)
    bv = (bv0, bv1, bv2)
    wv = av
    hb = (h0, h1, h2)
    sg = (sg0, sg1, sg2)
    ss = (ss0, ss1, ss2)

    cid = lax.axis_index("c")
    sid = lax.axis_index("s")
    wid = cid * NS + sid

    for q in range(D // 16):
        zbuf[0, pl.ds(16 * q, 16)] = jnp.zeros((16,), _f32)
    for r in range(1, 8):
        zbuf[r, :] = zbuf[0, :]
    rbase = sid * RPT
    for t in range(RPT // 8):
        pltpu.sync_copy(zbuf, num_sh.at[pl.ds(rbase + 8 * t, 8), :])
    dbase = sid * (DNP // NS)
    for t in range(DNP // NS // D):
        pltpu.sync_copy(zbuf.at[0], den_sh.at[pl.ds(dbase + D * t, D)])
    plsc.subcore_barrier()

    ebase = wid * EPT
    pltpu.sync_copy(pk_hbm.at[pl.ds(ebase, EPT)], pk_all)

    def fire_gathers(i, b):
        off = i * CHUNK
        for q in range(CHUNK // 16):
            p = pk_all[pl.ds(off + 16 * q, 16)]
            rv[b][pl.ds(16 * q, 16)] = p >> 14
            cv[b][pl.ds(16 * q, 16)] = p & 16383
        pltpu.async_copy(ar_hbm.at[rv[b]], av[b], sg[b])
        pltpu.async_copy(ac_hbm.at[cv[b]], bv[b], sg[b])
        pltpu.async_copy(hx_hbm.at[cv[b]], hb[b], sg[b])

    def wait_gathers(b):
        pltpu.make_async_copy(ar_hbm.at[rv[b]], av[b], sg[b]).wait()
        pltpu.make_async_copy(ac_hbm.at[cv[b]], bv[b], sg[b]).wait()
        pltpu.make_async_copy(hx_hbm.at[cv[b]], hb[b], sg[b]).wait()

    def wait_scatter(b):
        pltpu.make_async_copy(hb[b], num_sh.at[rv[b]], ss[b]).wait()
        pltpu.make_async_copy(wv[b], den_sh.at[rv[b]], ss[b]).wait()

    def compute(b):
        def grp(g, c):
            s = av[b][pl.ds(g * 16, 16)] + bv[b][pl.ds(g * 16, 16)]
            s = jnp.where(s > 0, s, ALPHA * s)
            w = jnp.exp(s)
            wv[b][pl.ds(g * 16, 16)] = w
            for l in range(16):
                wvec = jnp.full((16,), w[l], _f32)
                j = g * 16 + l
                for q in range(8):
                    hb[b][j, pl.ds(16 * q, 16)] = hb[b][j, pl.ds(16 * q, 16)] * wvec
            return c

        lax.fori_loop(0, CHUNK // 16, grp, 0)

    def slot(i, b, wait_prev, fire):
        wait_gathers(b)
        compute(b)
        if wait_prev:
            wait_scatter((b + 2) % 3)
        if fire:
            fire_gathers(i + 2, (b + 2) % 3)
        pltpu.async_copy(hb[b], num_sh.at[rv[b]], ss[b], add=True)
        pltpu.async_copy(wv[b], den_sh.at[rv[b]], ss[b], add=True)

    fire_gathers(0, 0)
    fire_gathers(1, 1)
    slot(0, 0, False, True)

    def loop_body(kk, c):
        i = 3 * kk + 1
        slot(i, 1, True, True)
        slot(i + 1, 2, True, True)
        slot(i + 2, 0, True, True)
        return c

    lax.fori_loop(0, 34, loop_body, 0)
    slot(103, 1, True, False)
    slot(104, 2, True, False)
    wait_scatter(2)
    plsc.subcore_barrier()

    pltpu.sync_copy(num_sh.at[pl.ds(rbase, RPT), :], num_hbm.at[cid, pl.ds(rbase, RPT), :])
    pltpu.sync_copy(den_sh.at[pl.ds(dbase, DNP // NS)],
                    den_hbm.at[pl.ds(cid * DNP + dbase, DNP // NS)])


def _sc_call(hx, ar, ac, pk):
    mesh = plsc.VectorSubcoreMesh(core_axis_name="c", subcore_axis_name="s")
    idx_t = lambda: pltpu.VMEM((CHUNK,), jnp.int32)
    val_t = lambda: pltpu.VMEM((CHUNK,), _f32)
    row_t = lambda: pltpu.VMEM((CHUNK, D), _f32)
    return pl.kernel(
        _sc_body,
        out_type=(
            jax.ShapeDtypeStruct((NC, NP, D), _f32),
            jax.ShapeDtypeStruct((NC * DNP,), _f32),
        ),
        mesh=mesh,
        scratch_types=[
            pltpu.VMEM_SHARED((NP, D), _f32),
            pltpu.VMEM_SHARED((DNP,), _f32),
            pltpu.VMEM((EPT,), jnp.int32),
            idx_t(), idx_t(), idx_t(), idx_t(), idx_t(), idx_t(),
            val_t(), val_t(), val_t(), val_t(), val_t(), val_t(),
            row_t(), row_t(), row_t(),
            pltpu.VMEM((8, D), _f32),
            pltpu.SemaphoreType.DMA, pltpu.SemaphoreType.DMA,
            pltpu.SemaphoreType.DMA, pltpu.SemaphoreType.DMA,
            pltpu.SemaphoreType.DMA, pltpu.SemaphoreType.DMA,
        ],
    )(hx, ar, ac, pk)


def _comb_body(num_ref, den_ref, out_ref):
    n = num_ref[0] + num_ref[1]
    d = den_ref[0] + den_ref[1]
    out_ref[...] = jnp.where(d[:, None] > 0, n / d[:, None], 0.0)


def _comb_call(num2, den2):
    return pl.pallas_call(
        _comb_body,
        grid=(NP // _CBLK,),
        in_specs=[
            pl.BlockSpec((NC, _CBLK, D), lambda i: (0, i, 0)),
            pl.BlockSpec((NC, _CBLK), lambda i: (0, i)),
        ],
        out_specs=pl.BlockSpec((_CBLK, D), lambda i: (i, 0)),
        out_shape=jax.ShapeDtypeStruct((NP, D), _f32),
    )(num2, den2)


@jax.jit
def kernel(x, edge_index, W, a_row, a_col):
    ei = edge_index.astype(jnp.int32)
    pk0 = (ei[0] << 14) | ei[1]
    pk = jnp.pad(pk0.reshape(NW, E // NW), ((0, 0), (0, EPT - E // NW)),
                 constant_values=DROW << 14).reshape(-1)
    hx, ar, ac = _mm_call(x, W, a_row, a_col)
    num2, den2 = _sc_call(hx, ar, ac, pk)
    return _comb_call(num2, den2.reshape(NC, DNP))[:N]


# speedup vs baseline: 1.6419x; 1.6419x over previous
import functools

import jax
import jax.numpy as jnp
from jax import lax
from jax.experimental import pallas as pl
from jax.experimental.pallas import tpu as pltpu
from jax.experimental.pallas import tpu_sc as plsc

N = 10000
E = 320000
D = 128
ALPHA = 0.2

NP = 10240
NC, NS = 2, 16
NW = NC * NS
EPT = E // NW
CHUNK = 80
NCHUNK = EPT // CHUNK
RPT = NP // NS

_f32 = jnp.float32


def _mm_body(x_ref, w_ref, arow_ref, acol_ref, h_ref, ar_ref, ac_ref):
    h = jnp.dot(x_ref[...], w_ref[...], preferred_element_type=_f32)
    h_ref[...] = h
    ar_ref[...] = jnp.dot(h, arow_ref[...], preferred_element_type=_f32)[:, 0]
    ac_ref[...] = jnp.dot(h, acol_ref[...], preferred_element_type=_f32)[:, 0]


_MMBLK = 1024


_XBLK = 1000


def _mm_call(x, W, a_row, a_col):
    return pl.pallas_call(
        _mm_body,
        grid=(NP // _MMBLK,),
        in_specs=[
            pl.BlockSpec((_MMBLK, D), lambda i: (i, 0)),
            pl.BlockSpec((D, D), lambda i: (0, 0)),
            pl.BlockSpec((D, 1), lambda i: (0, 0)),
            pl.BlockSpec((D, 1), lambda i: (0, 0)),
        ],
        out_specs=[
            pl.BlockSpec((_MMBLK, D), lambda i: (i, 0)),
            pl.BlockSpec((_MMBLK,), lambda i: (i,)),
            pl.BlockSpec((_MMBLK,), lambda i: (i,)),
        ],
        out_shape=[
            jax.ShapeDtypeStruct((NP, D), _f32),
            jax.ShapeDtypeStruct((NP,), _f32),
            jax.ShapeDtypeStruct((NP,), _f32),
        ],
    )(x, W, a_row, a_col)


def _sc_body(hx_hbm, ar_hbm, ac_hbm, pk_hbm,
             num_hbm, den_hbm,
             num_sh, den_sh,
             pk_all,
             rv0, rv1, rv2, cv0, cv1, cv2,
             av0, av1, av2, bv0, bv1, bv2,
             wv0, wv1, wv2,
             h0, h1, h2, zbuf,
             sg0, sg1, sg2, ss0, ss1, ss2):
    rv = (rv0, rv1, rv2)
    cv = (cv0, cv1, cv2)
    av = (av0, av1, av2)
    bv = (bv0, bv1, bv2)
    wv = (wv0, wv1, wv2)
    hb = (h0, h1, h2)
    sg = (sg0, sg1, sg2)
    ss = (ss0, ss1, ss2)

    cid = lax.axis_index("c")
    sid = lax.axis_index("s")
    wid = cid * NS + sid

    for q in range(D // 16):
        zbuf[0, pl.ds(16 * q, 16)] = jnp.zeros((16,), _f32)
    for r in range(1, 16):
        zbuf[r, :] = zbuf[0, :]
    rbase = sid * RPT
    for t in range(RPT // 16):
        pltpu.sync_copy(zbuf, num_sh.at[pl.ds(rbase + 16 * t, 16), :])
    for t in range(RPT // D):
        pltpu.sync_copy(zbuf.at[0], den_sh.at[pl.ds(rbase + D * t, D)])
    plsc.subcore_barrier()

    ebase = wid * EPT
    pltpu.sync_copy(pk_hbm.at[pl.ds(ebase, EPT + CHUNK)], pk_all)

    def fire_gathers(i, b):
        off = i * CHUNK
        for q in range(CHUNK // 16):
            p = pk_all[pl.ds(off + 16 * q, 16)]
            rv[b][pl.ds(16 * q, 16)] = p >> 14
            cv[b][pl.ds(16 * q, 16)] = p & 16383
        pltpu.async_copy(ar_hbm.at[rv[b]], av[b], sg[b])
        pltpu.async_copy(ac_hbm.at[cv[b]], bv[b], sg[b])
        pltpu.async_copy(hx_hbm.at[cv[b]], hb[b], sg[b])

    def wait_gathers(b):
        pltpu.make_async_copy(ar_hbm.at[rv[b]], av[b], sg[b]).wait()
        pltpu.make_async_copy(ac_hbm.at[cv[b]], bv[b], sg[b]).wait()
        pltpu.make_async_copy(hx_hbm.at[cv[b]], hb[b], sg[b]).wait()

    def wait_scatter(b):
        pltpu.make_async_copy(hb[b], num_sh.at[rv[b]], ss[b]).wait()
        pltpu.make_async_copy(wv[b], den_sh.at[rv[b]], ss[b]).wait()

    def compute(b):
        def grp(g, c):
            s = av[b][pl.ds(g * 16, 16)] + bv[b][pl.ds(g * 16, 16)]
            s = jnp.where(s > 0, s, ALPHA * s)
            w = jnp.exp(s)
            wv[b][pl.ds(g * 16, 16)] = w
            for l in range(16):
                wvec = jnp.full((16,), w[l], _f32)
                j = g * 16 + l
                for q in range(8):
                    hb[b][j, pl.ds(16 * q, 16)] = hb[b][j, pl.ds(16 * q, 16)] * wvec
            return c

        lax.fori_loop(0, CHUNK // 16, grp, 0)

    def slot(i, b, wait_prev, fire):
        wait_gathers(b)
        compute(b)
        if wait_prev:
            wait_scatter((b + 2) % 3)
        if fire:
            fire_gathers(i + 2, (b + 2) % 3)
        pltpu.async_copy(hb[b], num_sh.at[rv[b]], ss[b], add=True)
        pltpu.async_copy(wv[b], den_sh.at[rv[b]], ss[b], add=True)

    fire_gathers(0, 0)
    fire_gathers(1, 1)
    slot(0, 0, False, True)

    def loop_body(kk, c):
        i = 3 * kk + 1
        slot(i, 1, True, True)
        slot(i + 1, 2, True, True)
        slot(i + 2, 0, True, True)
        return c

    lax.fori_loop(0, 41, loop_body, 0)
    slot(124, 1, True, False)
    wait_scatter(1)
    wait_gathers(2)
    plsc.subcore_barrier()

    pltpu.sync_copy(num_sh.at[pl.ds(rbase, RPT), :], num_hbm.at[cid, pl.ds(rbase, RPT), :])
    pltpu.sync_copy(den_sh.at[pl.ds(rbase, RPT)], den_hbm.at[cid, pl.ds(rbase, RPT)])


def _sc_call(hx, ar, ac, pk):
    mesh = plsc.VectorSubcoreMesh(core_axis_name="c", subcore_axis_name="s")
    idx_t = lambda: pltpu.VMEM((CHUNK,), jnp.int32)
    val_t = lambda: pltpu.VMEM((CHUNK,), _f32)
    row_t = lambda: pltpu.VMEM((CHUNK, D), _f32)
    return pl.kernel(
        _sc_body,
        out_type=(
            jax.ShapeDtypeStruct((NC, NP, D), _f32),
            jax.ShapeDtypeStruct((NC, NP), _f32),
        ),
        mesh=mesh,
        scratch_types=[
            pltpu.VMEM_SHARED((NP, D), _f32),
            pltpu.VMEM_SHARED((NP,), _f32),
            pltpu.VMEM((EPT + CHUNK,), jnp.int32),
            idx_t(), idx_t(), idx_t(), idx_t(), idx_t(), idx_t(),
            val_t(), val_t(), val_t(), val_t(), val_t(), val_t(),
            val_t(), val_t(), val_t(),
            row_t(), row_t(), row_t(),
            pltpu.VMEM((16, D), _f32),
            pltpu.SemaphoreType.DMA, pltpu.SemaphoreType.DMA,
            pltpu.SemaphoreType.DMA, pltpu.SemaphoreType.DMA,
            pltpu.SemaphoreType.DMA, pltpu.SemaphoreType.DMA,
        ],
    )(hx, ar, ac, pk)


def _comb_body(num_ref, den_ref, out_ref):
    n = num_ref[0] + num_ref[1]
    d = den_ref[0] + den_ref[1]
    out_ref[...] = jnp.where(d[:, None] > 0, n / d[:, None], 0.0)


def _comb_call(num2, den2):
    return pl.pallas_call(
        _comb_body,
        grid=(NP // _MMBLK,),
        in_specs=[
            pl.BlockSpec((NC, _MMBLK, D), lambda i: (0, i, 0)),
            pl.BlockSpec((NC, _MMBLK), lambda i: (0, i)),
        ],
        out_specs=pl.BlockSpec((_MMBLK, D), lambda i: (i, 0)),
        out_shape=jax.ShapeDtypeStruct((NP, D), _f32),
    )(num2, den2)


@jax.jit
def kernel(x, edge_index, W, a_row, a_col):
    ei = edge_index.astype(jnp.int32)
    pk = jnp.pad((ei[0] << 14) | ei[1], (0, CHUNK))
    hx, ar, ac = _mm_call(x, W, a_row, a_col)
    num2, den2 = _sc_call(hx, ar, ac, pk)
    return _comb_call(num2, den2)[:N]
